# baseline (speedup 1.0000x reference)
import jax
import jax.numpy as jnp
from jax.experimental import pallas as pl
from jax.experimental.pallas import tpu as pltpu

_BM = 400
_SLOTS = 3
_XC = 5000


def _fused_body(wt_ref, b_ref, prelu_ref, wb_ref, bias_ref,
                invn_ref, x1_ref, x2_ref, adj_ref, out_ref,
                y_ref, h_ref, csum_ref, abuf_ref, xbuf_ref, sems, xsem):
    hdim = wt_ref.shape[1]
    n = adj_ref.shape[0]
    nblk = n // _BM

    def _adj_copy(blk, slot):
        return pltpu.make_async_copy(
            adj_ref.at[pl.ds(blk * _BM, _BM), :], abuf_ref.at[slot],
            sems.at[slot])

    for s in range(_SLOTS):
        _adj_copy(s, s).start()

    wtb = wt_ref[:].astype(jnp.bfloat16)
    for k, xref in enumerate((x1_ref, x1_ref, x2_ref, x2_ref)):
        half = k % 2
        cp = pltpu.make_async_copy(
            xref.at[pl.ds(half * _XC, _XC), :], xbuf_ref, xsem)
        cp.start()
        cp.wait()
        yk = jnp.dot(xbuf_ref[:].astype(jnp.bfloat16), wtb,
                     preferred_element_type=jnp.float32)
        r0 = half * _XC
        c0 = (k // 2) * hdim
        y_ref[pl.ds(r0, _XC), pl.ds(c0, hdim)] = (
            yk + b_ref[:]).astype(jnp.bfloat16)
    csum_ref[:] = jnp.zeros_like(csum_ref)

    p = prelu_ref[0, 0]

    def _block(ib, carry):
        slot = jax.lax.rem(ib, _SLOTS)
        _adj_copy(ib, slot).wait()
        acc = jnp.dot(abuf_ref[slot].astype(jnp.bfloat16), y_ref[:],
                      preferred_element_type=jnp.float32)
        hj = jnp.where(acc >= 0, acc, p * acc)
        h_ref[pl.ds(ib * _BM, _BM), :] = hj.astype(jnp.bfloat16)
        csum_ref[0, :] = csum_ref[0, :] + jnp.sum(hj[:, :hdim], axis=0)

        @pl.when(ib + _SLOTS < nblk)
        def _():
            _adj_copy(ib + _SLOTS, slot).start()
        return carry

    jax.lax.fori_loop(0, nblk, _block, 0)

    s = jax.nn.sigmoid(csum_ref[:] * invn_ref[0, 0])
    u = jnp.dot(s, wb_ref[:].T, preferred_element_type=jnp.float32)
    bias = bias_ref[0, 0]
    ub = u.astype(jnp.bfloat16)
    zero = jnp.zeros_like(ub)
    u2 = jnp.concatenate(
        [jnp.concatenate([ub, zero], axis=1),
         jnp.concatenate([zero, ub], axis=1)], axis=0)
    r = jax.lax.dot_general(u2, h_ref[:],
                            (((1,), (1,)), ((), ())),
                            preferred_element_type=jnp.float32)
    out_ref[0, :n] = r[0, :] + bias
    out_ref[0, n:] = r[1, :] + bias


def kernel(x_1, x_2, adj, W, b, prelu_w, Wb, bias_b):
    n = adj.shape[0]
    f_in = x_1.shape[2]
    h_dim = W.shape[0]

    x1 = x_1[0]
    x2 = x_2[0]
    wt = W.T
    b2 = b.reshape(1, h_dim)
    prelu2 = prelu_w.reshape(1, 1)
    wb2 = Wb[0]
    bias2 = bias_b.reshape(1, 1)
    inv_n = jnp.full((1, 1), 1.0 / n, dtype=jnp.float32)

    logits = pl.pallas_call(
        _fused_body,
        grid=(1,),
        in_specs=[
            pl.BlockSpec((f_in, h_dim), lambda i: (0, 0)),
            pl.BlockSpec((1, h_dim), lambda i: (0, 0)),
            pl.BlockSpec((1, 1), lambda i: (0, 0)),
            pl.BlockSpec((h_dim, h_dim), lambda i: (0, 0)),
            pl.BlockSpec((1, 1), lambda i: (0, 0)),
            pl.BlockSpec((1, 1), lambda i: (0, 0)),
            pl.BlockSpec(memory_space=pltpu.HBM),
            pl.BlockSpec(memory_space=pltpu.HBM),
            pl.BlockSpec(memory_space=pltpu.HBM),
        ],
        out_specs=pl.BlockSpec((1, 2 * n), lambda i: (0, 0)),
        out_shape=jax.ShapeDtypeStruct((1, 2 * n), jnp.float32),
        scratch_shapes=[
            pltpu.VMEM((n, 2 * h_dim), jnp.bfloat16),
            pltpu.VMEM((n, 2 * h_dim), jnp.bfloat16),
            pltpu.VMEM((1, h_dim), jnp.float32),
            pltpu.VMEM((_SLOTS, _BM, n), jnp.float32),
            pltpu.VMEM((_XC, f_in), jnp.float32),
            pltpu.SemaphoreType.DMA((_SLOTS,)),
            pltpu.SemaphoreType.DMA,
        ],
        compiler_params=pltpu.CompilerParams(
            dimension_semantics=(pltpu.ARBITRARY,),
            vmem_limit_bytes=64 * 1024 * 1024),
    )(wt, b2, prelu2, wb2, bias2, inv_n, x1, x2, adj)

    return logits


# speedup vs baseline: 1.0219x; 1.0219x over previous
import jax
import jax.numpy as jnp
from jax.experimental import pallas as pl
from jax.experimental.pallas import tpu as pltpu

_BM = 400
_SLOTS = 3
_XC = 5000


def _fused_body(wt_ref, b_ref, prelu_ref, wb_ref, bias_ref,
                invn_ref, x1_ref, x2_ref, adj_ref, out_ref,
                y_ref, h_ref, csum_ref, abuf_ref, xbuf_ref, sems, xsem):
    hdim = wt_ref.shape[1]
    n = adj_ref.shape[0]
    nblk = n // _BM

    def _adj_copy(blk, slot):
        return pltpu.make_async_copy(
            adj_ref.at[pl.ds(blk * _BM, _BM), :], abuf_ref.at[slot],
            sems.at[slot])

    wtb = wt_ref[:].astype(jnp.bfloat16)
    for k, xref in enumerate((x1_ref, x1_ref, x2_ref, x2_ref)):
        half = k % 2
        cp = pltpu.make_async_copy(
            xref.at[pl.ds(half * _XC, _XC), :], xbuf_ref, xsem)
        cp.start()
        cp.wait()
        yk = jnp.dot(xbuf_ref[:].astype(jnp.bfloat16), wtb,
                     preferred_element_type=jnp.float32)
        r0 = half * _XC
        c0 = (k // 2) * hdim
        y_ref[pl.ds(r0, _XC), pl.ds(c0, hdim)] = (
            yk + b_ref[:]).astype(jnp.bfloat16)
    csum_ref[:] = jnp.zeros_like(csum_ref)

    for s in range(_SLOTS):
        _adj_copy(s, s).start()

    p = prelu_ref[0, 0]

    def _block(ib, carry):
        slot = jax.lax.rem(ib, _SLOTS)
        _adj_copy(ib, slot).wait()
        acc = jnp.dot(abuf_ref[slot].astype(jnp.bfloat16), y_ref[:],
                      preferred_element_type=jnp.float32)
        hj = jnp.where(acc >= 0, acc, p * acc)
        h_ref[pl.ds(ib * _BM, _BM), :] = hj.astype(jnp.bfloat16)
        csum_ref[0, :] = csum_ref[0, :] + jnp.sum(hj[:, :hdim], axis=0)

        @pl.when(ib + _SLOTS < nblk)
        def _():
            _adj_copy(ib + _SLOTS, slot).start()
        return carry

    jax.lax.fori_loop(0, nblk, _block, 0)

    s = jax.nn.sigmoid(csum_ref[:] * invn_ref[0, 0])
    u = jnp.dot(s, wb_ref[:].T, preferred_element_type=jnp.float32)
    bias = bias_ref[0, 0]
    ub = u.astype(jnp.bfloat16)
    zero = jnp.zeros_like(ub)
    u2 = jnp.concatenate(
        [jnp.concatenate([ub, zero], axis=1),
         jnp.concatenate([zero, ub], axis=1)], axis=0)
    r = jax.lax.dot_general(u2, h_ref[:],
                            (((1,), (1,)), ((), ())),
                            preferred_element_type=jnp.float32)
    out_ref[0, :n] = r[0, :] + bias
    out_ref[0, n:] = r[1, :] + bias


def kernel(x_1, x_2, adj, W, b, prelu_w, Wb, bias_b):
    n = adj.shape[0]
    f_in = x_1.shape[2]
    h_dim = W.shape[0]

    x1 = x_1[0]
    x2 = x_2[0]
    wt = W.T
    b2 = b.reshape(1, h_dim)
    prelu2 = prelu_w.reshape(1, 1)
    wb2 = Wb[0]
    bias2 = bias_b.reshape(1, 1)
    inv_n = jnp.full((1, 1), 1.0 / n, dtype=jnp.float32)

    logits = pl.pallas_call(
        _fused_body,
        grid=(1,),
        in_specs=[
            pl.BlockSpec((f_in, h_dim), lambda i: (0, 0)),
            pl.BlockSpec((1, h_dim), lambda i: (0, 0)),
            pl.BlockSpec((1, 1), lambda i: (0, 0)),
            pl.BlockSpec((h_dim, h_dim), lambda i: (0, 0)),
            pl.BlockSpec((1, 1), lambda i: (0, 0)),
            pl.BlockSpec((1, 1), lambda i: (0, 0)),
            pl.BlockSpec(memory_space=pltpu.HBM),
            pl.BlockSpec(memory_space=pltpu.HBM),
            pl.BlockSpec(memory_space=pltpu.HBM),
        ],
        out_specs=pl.BlockSpec((1, 2 * n), lambda i: (0, 0)),
        out_shape=jax.ShapeDtypeStruct((1, 2 * n), jnp.float32),
        scratch_shapes=[
            pltpu.VMEM((n, 2 * h_dim), jnp.bfloat16),
            pltpu.VMEM((n, 2 * h_dim), jnp.bfloat16),
            pltpu.VMEM((1, h_dim), jnp.float32),
            pltpu.VMEM((_SLOTS, _BM, n), jnp.float32),
            pltpu.VMEM((_XC, f_in), jnp.float32),
            pltpu.SemaphoreType.DMA((_SLOTS,)),
            pltpu.SemaphoreType.DMA,
        ],
        compiler_params=pltpu.CompilerParams(
            dimension_semantics=(pltpu.ARBITRARY,),
            vmem_limit_bytes=64 * 1024 * 1024),
    )(wt, b2, prelu2, wb2, bias2, inv_n, x1, x2, adj)

    return logits


# speedup vs baseline: 1.1037x; 1.0801x over previous
import jax
import jax.numpy as jnp
from jax.experimental import pallas as pl
from jax.experimental.pallas import tpu as pltpu


def _fused_body(x1_ref, x2_ref, wt_ref, b_ref, prelu_ref, wb_ref, bias_ref,
                invn_ref, adj_ref, out_ref,
                y_ref, h_ref, csum_ref):
    i = pl.program_id(0)
    g = pl.num_programs(0)
    hdim = wt_ref.shape[1]
    m = adj_ref.shape[0]

    @pl.when(i == 0)
    def _linear():
        y1 = jnp.dot(x1_ref[:].astype(jnp.bfloat16),
                     wt_ref[:].astype(jnp.bfloat16),
                     preferred_element_type=jnp.float32)
        y2 = jnp.dot(x2_ref[:].astype(jnp.bfloat16),
                     wt_ref[:].astype(jnp.bfloat16),
                     preferred_element_type=jnp.float32)
        y_ref[:, :hdim] = (y1 + b_ref[:]).astype(jnp.bfloat16)
        y_ref[:, hdim:] = (y2 + b_ref[:]).astype(jnp.bfloat16)
        csum_ref[:] = jnp.zeros_like(csum_ref)

    p = prelu_ref[0, 0]
    acc = jnp.dot(adj_ref[:].astype(jnp.bfloat16), y_ref[:],
                  preferred_element_type=jnp.float32)
    hj = jnp.where(acc >= 0, acc, p * acc)
    h_ref[pl.ds(i * m, m), :] = hj.astype(jnp.bfloat16)
    csum_ref[0, :] = csum_ref[0, :] + jnp.sum(hj[:, :hdim], axis=0)

    @pl.when(i == g - 1)
    def _score():
        n = h_ref.shape[0]
        s = jax.nn.sigmoid(csum_ref[:] * invn_ref[0, 0])
        u = jnp.dot(s, wb_ref[:].T, preferred_element_type=jnp.float32)
        bias = bias_ref[0, 0]
        ub = u.astype(jnp.bfloat16)
        zero = jnp.zeros_like(ub)
        u2 = jnp.concatenate(
            [jnp.concatenate([ub, zero], axis=1),
             jnp.concatenate([zero, ub], axis=1)], axis=0)
        r = jax.lax.dot_general(u2, h_ref[:],
                                (((1,), (1,)), ((), ())),
                                preferred_element_type=jnp.float32)
        out_ref[0, :n] = r[0, :] + bias
        out_ref[0, n:] = r[1, :] + bias


def kernel(x_1, x_2, adj, W, b, prelu_w, Wb, bias_b):
    n = adj.shape[0]
    f_in = x_1.shape[2]
    h_dim = W.shape[0]

    x1 = x_1[0]
    x2 = x_2[0]
    wt = W.T
    b2 = b.reshape(1, h_dim)
    prelu2 = prelu_w.reshape(1, 1)
    wb2 = Wb[0]
    bias2 = bias_b.reshape(1, 1)
    inv_n = jnp.full((1, 1), 1.0 / n, dtype=jnp.float32)

    bm = 400
    g = n // bm

    logits = pl.pallas_call(
        _fused_body,
        grid=(g,),
        in_specs=[
            pl.BlockSpec((n, f_in), lambda i: (0, 0)),
            pl.BlockSpec((n, f_in), lambda i: (0, 0)),
            pl.BlockSpec((f_in, h_dim), lambda i: (0, 0)),
            pl.BlockSpec((1, h_dim), lambda i: (0, 0)),
            pl.BlockSpec((1, 1), lambda i: (0, 0)),
            pl.BlockSpec((h_dim, h_dim), lambda i: (0, 0)),
            pl.BlockSpec((1, 1), lambda i: (0, 0)),
            pl.BlockSpec((1, 1), lambda i: (0, 0)),
            pl.BlockSpec((bm, n), lambda i: (i, 0)),
        ],
        out_specs=pl.BlockSpec((1, 2 * n), lambda i: (0, 0)),
        out_shape=jax.ShapeDtypeStruct((1, 2 * n), jnp.float32),
        scratch_shapes=[
            pltpu.VMEM((n, 2 * h_dim), jnp.bfloat16),
            pltpu.VMEM((n, 2 * h_dim), jnp.bfloat16),
            pltpu.VMEM((1, h_dim), jnp.float32),
        ],
        compiler_params=pltpu.CompilerParams(
            dimension_semantics=(pltpu.ARBITRARY,),
            vmem_limit_bytes=64 * 1024 * 1024),
    )(x1, x2, wt, b2, prelu2, wb2, bias2, inv_n, adj)

    return logits
